# baseline (speedup 1.0000x reference)
import functools

import jax
import jax.numpy as jnp
from jax import lax
from jax.experimental import pallas as pl
from jax.experimental.pallas import tpu as pltpu
from jax.experimental.pallas import tpu_sc as plsc

EMB = 32
BATCH = 16384
TOTAL = 2 * BATCH
NC, NS = 2, 16
NW = NC * NS
CHUNK = 128
N_CHUNKS_TOTAL = TOTAL // CHUNK
N_CHUNKS_W = N_CHUNKS_TOTAL // NW
NBUF = 2

PACK = 4
TCOLS = 8192
TSUB = TCOLS // PACK
TGRID = 123
N_LINES = TGRID * TSUB

BLK = 512


def _pack_body(tt_ref, out_ref):
    x = tt_ref[...]
    cols = [x[:, q * TSUB:(q + 1) * TSUB].T for q in range(PACK)]
    out_ref[...] = jnp.concatenate(cols, axis=1)


def _pack_table(table_t):
    return pl.pallas_call(
        _pack_body,
        grid=(TGRID,),
        in_specs=[pl.BlockSpec((EMB, TCOLS), lambda i: (0, i))],
        out_specs=pl.BlockSpec((TSUB, PACK * EMB), lambda i: (i, 0)),
        out_shape=jax.ShapeDtypeStruct((N_LINES, PACK * EMB), jnp.float32),
    )(table_t)


@functools.lru_cache(maxsize=None)
def _make_sc_gather():
    @functools.partial(
        pl.kernel,
        out_type=jax.ShapeDtypeStruct(
            (N_CHUNKS_TOTAL, CHUNK, PACK * EMB), jnp.float32),
        mesh=plsc.VectorSubcoreMesh(core_axis_name="c", subcore_axis_name="s"),
        scratch_types=[
            pltpu.VMEM((N_CHUNKS_W, CHUNK), jnp.int32),
            pltpu.VMEM((NBUF, CHUNK, PACK * EMB), jnp.float32),
            pltpu.SemaphoreType.DMA,
            pltpu.SemaphoreType.DMA,
            pltpu.SemaphoreType.DMA,
        ],
    )
    def _sc_gather(packed_hbm, lidx_hbm, out_hbm, idx_v, lines_v,
                   gsem0, gsem1, osem):
        gsems = [gsem0, gsem1]
        wid = lax.axis_index("s") * NC + lax.axis_index("c")
        base = wid * N_CHUNKS_W
        pltpu.sync_copy(lidx_hbm.at[pl.ds(base, N_CHUNKS_W)], idx_v)
        gathers = [None] * N_CHUNKS_W
        outs = [None] * N_CHUNKS_W
        for j in range(NBUF):
            gathers[j] = pltpu.async_copy(
                packed_hbm.at[idx_v.at[j]], lines_v.at[j % NBUF],
                gsems[j % NBUF])
        for j in range(N_CHUNKS_W):
            gathers[j].wait()
            outs[j] = pltpu.async_copy(
                lines_v.at[j % NBUF], out_hbm.at[base + j], osem)
            nxt = j + NBUF
            if nxt < N_CHUNKS_W:
                outs[j].wait()
                gathers[nxt] = pltpu.async_copy(
                    packed_hbm.at[idx_v.at[nxt]], lines_v.at[nxt % NBUF],
                    gsems[nxt % NBUF])
        for j in range(N_CHUNKS_W - NBUF, N_CHUNKS_W):
            if outs[j] is not None:
                outs[j].wait()

    return _sc_gather


def _mlp_body(ids_ref, t1_ref, t2_ref, w1_ref, b1_ref,
              w2_ref, b2_ref, w3_ref, b3_ref, out_ref):
    ids = ids_ref[...]
    i1 = ids[:, 0:1].astype(jnp.int32)
    i2 = ids[:, 1:2].astype(jnp.int32)
    q1 = (i1 >> 11) & (PACK - 1)
    q2 = (i2 >> 11) & (PACK - 1)
    f1 = jnp.zeros((BLK, EMB), jnp.float32)
    f2 = jnp.zeros((BLK, EMB), jnp.float32)
    for q in range(PACK):
        f1 = jnp.where(q1 == q, t1_ref[:, q * EMB:(q + 1) * EMB], f1)
        f2 = jnp.where(q2 == q, t2_ref[:, q * EMB:(q + 1) * EMB], f2)
    score = ids[:, 2:3]
    f = jnp.concatenate([f1, f2, score], axis=1)
    dn = (((1,), (0,)), ((), ()))
    hp = lax.dot_general(f, w1_ref[...], dn)
    h = jnp.maximum(hp + b1_ref[...], 0.0)
    hp2 = lax.dot_general(h, w2_ref[...], dn)
    h2 = jnp.maximum(hp2 + b2_ref[...], 0.0)
    o = lax.dot_general(h2, w3_ref[...], dn) + b3_ref[...]
    out_ref[...] = jax.nn.sigmoid(o)


def _mlp(ids, lines, W1, b1, W2, b2, W3, b3):
    nblk = BATCH // BLK
    full = lambda shape: pl.BlockSpec(shape, lambda i: (0, 0))
    return pl.pallas_call(
        _mlp_body,
        grid=(nblk,),
        in_specs=[
            pl.BlockSpec((BLK, 3), lambda i: (i, 0)),
            pl.BlockSpec((BLK, PACK * EMB), lambda i: (i, 0)),
            pl.BlockSpec((BLK, PACK * EMB), lambda i: (i + nblk, 0)),
            full((2 * EMB + 1, 64)),
            full((1, 64)),
            full((64, 32)),
            full((1, 32)),
            full((32, 1)),
            full((1, 1)),
        ],
        out_specs=pl.BlockSpec((BLK, 1), lambda i: (i, 0)),
        out_shape=jax.ShapeDtypeStruct((BATCH, 1), jnp.float32),
    )(ids, lines, lines, W1, b1, W2, b2, W3, b3)


def kernel(idsTensor, table, W1, b1, W2, b2, W3, b3):
    idx = idsTensor[:, :2].astype(jnp.int32)
    lidx = ((idx >> 13) << 11) | (idx & (TSUB - 1))
    lidx = lidx.T.reshape(N_CHUNKS_TOTAL, CHUNK)
    packed = _pack_table(table.T)
    lines = _make_sc_gather()(packed, lidx)
    lines = lines.reshape(TOTAL, PACK * EMB)
    out = _mlp(idsTensor, lines, W1,
               b1.reshape(1, 64), W2, b2.reshape(1, 32), W3,
               b3.reshape(1, 1))
    return out


# speedup vs baseline: 1.1075x; 1.1075x over previous
import functools

import jax
import jax.numpy as jnp
from jax import lax
from jax.experimental import pallas as pl
from jax.experimental.pallas import tpu as pltpu
from jax.experimental.pallas import tpu_sc as plsc

EMB = 32
BATCH = 16384
TOTAL = 2 * BATCH
NC, NS = 2, 16
NW = NC * NS
CHUNK = 128
N_CHUNKS_TOTAL = TOTAL // CHUNK
N_CHUNKS_W = N_CHUNKS_TOTAL // NW
NBUF = 2

N_ROWS = 1000000
TCOLS = 8192
TGRID = 123

BLK = 512


def _transpose_body(tt_ref, out_ref):
    out_ref[...] = tt_ref[...].T


def _pack_table(table_t):
    return pl.pallas_call(
        _transpose_body,
        grid=(TGRID,),
        in_specs=[pl.BlockSpec((EMB, TCOLS), lambda i: (0, i))],
        out_specs=pl.BlockSpec((TCOLS, EMB), lambda i: (i, 0)),
        out_shape=jax.ShapeDtypeStruct((N_ROWS, EMB), jnp.float32),
    )(table_t)


@functools.lru_cache(maxsize=None)
def _make_sc_gather():
    @functools.partial(
        pl.kernel,
        out_type=jax.ShapeDtypeStruct(
            (N_CHUNKS_TOTAL, CHUNK, EMB), jnp.float32),
        mesh=plsc.VectorSubcoreMesh(core_axis_name="c", subcore_axis_name="s"),
        scratch_types=[
            pltpu.VMEM((N_CHUNKS_W, CHUNK), jnp.int32),
            pltpu.VMEM((NBUF, CHUNK, EMB), jnp.float32),
            pltpu.SemaphoreType.DMA,
            pltpu.SemaphoreType.DMA,
            pltpu.SemaphoreType.DMA,
        ],
    )
    def _sc_gather(table_hbm, idx_hbm, out_hbm, idx_v, rows_v,
                   gsem0, gsem1, osem):
        gsems = [gsem0, gsem1]
        wid = lax.axis_index("s") * NC + lax.axis_index("c")
        base = wid * N_CHUNKS_W
        pltpu.sync_copy(idx_hbm.at[pl.ds(base, N_CHUNKS_W)], idx_v)
        outs = [None] * N_CHUNKS_W

        def fire_chunk(j):
            buf = rows_v.at[j % NBUF]

            @pl.loop(0, CHUNK // 16, unroll=2)
            def _(g):
                v = idx_v[j, pl.ds(g * 16, 16)]
                for l in range(16):
                    pltpu.async_copy(
                        table_hbm.at[pl.ds(v[l], 1)],
                        buf.at[pl.ds(g * 16 + l, 1)], gsems[j % NBUF])

        def drain_chunk(j):
            pltpu.make_async_copy(
                table_hbm.at[pl.ds(0, CHUNK)], rows_v.at[j % NBUF],
                gsems[j % NBUF]
            ).wait()

        for j in range(NBUF):
            fire_chunk(j)
        for j in range(N_CHUNKS_W):
            drain_chunk(j)
            outs[j] = pltpu.async_copy(
                rows_v.at[j % NBUF], out_hbm.at[base + j], osem)
            nxt = j + NBUF
            if nxt < N_CHUNKS_W:
                outs[j].wait()
                fire_chunk(nxt)
        for j in range(N_CHUNKS_W - NBUF, N_CHUNKS_W):
            if outs[j] is not None:
                outs[j].wait()

    return _sc_gather


def _mlp_body(ids_ref, t1_ref, t2_ref, w1_ref, b1_ref,
              w2_ref, b2_ref, w3_ref, b3_ref, out_ref):
    ids = ids_ref[...]
    score = ids[:, 2:3]
    f = jnp.concatenate([t1_ref[...], t2_ref[...], score], axis=1)
    dn = (((1,), (0,)), ((), ()))
    hp = lax.dot_general(f, w1_ref[...], dn)
    h = jnp.maximum(hp + b1_ref[...], 0.0)
    hp2 = lax.dot_general(h, w2_ref[...], dn)
    h2 = jnp.maximum(hp2 + b2_ref[...], 0.0)
    o = lax.dot_general(h2, w3_ref[...], dn) + b3_ref[...]
    out_ref[...] = jax.nn.sigmoid(o)


def _mlp(ids, gathered, W1, b1, W2, b2, W3, b3):
    nblk = BATCH // BLK
    full = lambda shape: pl.BlockSpec(shape, lambda i: (0, 0))
    return pl.pallas_call(
        _mlp_body,
        grid=(nblk,),
        in_specs=[
            pl.BlockSpec((BLK, 3), lambda i: (i, 0)),
            pl.BlockSpec((BLK, EMB), lambda i: (i, 0)),
            pl.BlockSpec((BLK, EMB), lambda i: (i + nblk, 0)),
            full((2 * EMB + 1, 64)),
            full((1, 64)),
            full((64, 32)),
            full((1, 32)),
            full((32, 1)),
            full((1, 1)),
        ],
        out_specs=pl.BlockSpec((BLK, 1), lambda i: (i, 0)),
        out_shape=jax.ShapeDtypeStruct((BATCH, 1), jnp.float32),
    )(ids, gathered, gathered, W1, b1, W2, b2, W3, b3)


def kernel(idsTensor, table, W1, b1, W2, b2, W3, b3):
    idx = idsTensor[:, :2].astype(jnp.int32)
    idx_all = idx.T.reshape(N_CHUNKS_TOTAL, CHUNK)
    table_rm = _pack_table(table.T)
    gathered = _make_sc_gather()(table_rm, idx_all)
    gathered = gathered.reshape(TOTAL, EMB)
    out = _mlp(idsTensor, gathered, W1,
               b1.reshape(1, 64), W2, b2.reshape(1, 32), W3,
               b3.reshape(1, 1))
    return out
